# Initial kernel scaffold; baseline (speedup 1.0000x reference)
import jax
import jax.numpy as jnp
from jax.experimental import pallas as pl


def kernel(x, edge_features, W1_e, b1_e, W2_e, b2_e, W1_n, b1_n, W2_n, b2_n, senders, receivers):
    raise NotImplementedError("write your pallas kernel here")


# speedup vs baseline: 3.8346x; 3.8346x over previous
import functools

import jax
import jax.numpy as jnp
from jax import lax
from jax.experimental import pallas as pl
from jax.experimental.pallas import tpu as pltpu
from jax.experimental.pallas import tpu_sc as plsc

_NC = 2
_NS = 16
_NW = _NC * _NS


def _pre_body(x_ref, ws_ref, wr_ref, xs_ref, xr_ref):
    x = x_ref[...]
    xs_ref[...] = jnp.dot(x, ws_ref[...], preferred_element_type=jnp.float32)
    xr_ref[...] = jnp.dot(x, wr_ref[...], preferred_element_type=jnp.float32)


def _pre(x, w_s, w_r):
    n, d = x.shape
    out = jax.ShapeDtypeStruct((n, d), jnp.float32)
    return pl.pallas_call(
        _pre_body,
        out_shape=(out, out),
    )(x, w_s, w_r)


def _gather(xs, xr, senders, receivers):
    e = senders.shape[0]
    d = xs.shape[1]
    blk = 400
    per_w = e // _NW
    chunks = per_w // blk
    assert per_w * _NW == e and chunks * blk == per_w

    mesh = plsc.VectorSubcoreMesh(core_axis_name="c", subcore_axis_name="s")
    out = jax.ShapeDtypeStruct((e, d), jnp.float32)

    @functools.partial(
        pl.kernel,
        out_type=(out, out),
        mesh=mesh,
        scratch_types=[
            pltpu.VMEM((blk,), jnp.int32),
            pltpu.VMEM((blk,), jnp.int32),
            pltpu.VMEM((blk, d), jnp.float32),
            pltpu.VMEM((blk, d), jnp.float32),
            pltpu.SemaphoreType.DMA,
            pltpu.SemaphoreType.DMA,
        ],
    )
    def k(xs_hbm, xr_hbm, s_hbm, r_hbm, gs_hbm, gr_hbm,
          sidx, ridx, srow, rrow, sem1, sem2):
        wid = lax.axis_index("s") * _NC + lax.axis_index("c")
        base = wid * per_w

        def body(ci, carry):
            off = base + ci * blk
            pltpu.sync_copy(s_hbm.at[pl.ds(off, blk)], sidx)
            pltpu.sync_copy(r_hbm.at[pl.ds(off, blk)], ridx)
            c1 = pltpu.async_copy(xs_hbm.at[sidx], srow, sem1)
            c2 = pltpu.async_copy(xr_hbm.at[ridx], rrow, sem2)
            c1.wait()
            c2.wait()
            pltpu.sync_copy(srow, gs_hbm.at[pl.ds(off, blk)])
            pltpu.sync_copy(rrow, gr_hbm.at[pl.ds(off, blk)])
            return carry

        lax.fori_loop(0, chunks, body, 0)

    return k(xs, xr, senders, receivers)


def _edge_body(gs_ref, gr_ref, ef_ref, w1_ref, b1_ref, w2_ref, b2_ref, out_ref):
    ef = ef_ref[...]
    h = gs_ref[...] + gr_ref[...] + b1_ref[...] + jnp.dot(
        ef, w1_ref[...], preferred_element_type=jnp.float32)
    h = jnp.maximum(h, 0.0)
    out_ref[...] = ef + b2_ref[...] + jnp.dot(
        h, w2_ref[...], preferred_element_type=jnp.float32)


def _edge(gs, gr, ef, w1, b1, w2, b2):
    e, d = ef.shape
    blk = 2000
    grid = e // blk
    assert grid * blk == e
    row = pl.BlockSpec((blk, d), lambda i: (i, 0))
    full = pl.BlockSpec((d, d), lambda i: (0, 0))
    vec = pl.BlockSpec((1, d), lambda i: (0, 0))
    return pl.pallas_call(
        _edge_body,
        grid=(grid,),
        in_specs=[row, row, row, full, vec, full, vec],
        out_specs=row,
        out_shape=jax.ShapeDtypeStruct((e, d), jnp.float32),
    )(gs, gr, ef, w1, b1, w2, b2)


def _scatter(new_e, receivers, zeros_nd):
    e, d = new_e.shape
    n = zeros_nd.shape[0]
    blk = 200
    per_core = e // _NC
    per_tile = per_core // _NS
    chunks = per_tile // blk
    rows_per_tile = n // _NS
    assert chunks * blk == per_tile and rows_per_tile * _NS == n
    assert rows_per_tile % 8 == 0

    mesh = plsc.VectorSubcoreMesh(core_axis_name="c", subcore_axis_name="s")
    out = jax.ShapeDtypeStruct((n, d), jnp.float32)

    @functools.partial(
        pl.kernel,
        out_type=(out, out),
        mesh=mesh,
        scratch_types=[
            pltpu.VMEM_SHARED((n, d), jnp.float32),
            pltpu.VMEM((blk,), jnp.int32),
            pltpu.VMEM((blk, d), jnp.float32),
        ],
    )
    def k(ne_hbm, r_hbm, z_hbm, a0_hbm, a1_hbm, shared, eidx, erow):
        c = lax.axis_index("c")
        s = lax.axis_index("s")
        r0 = s * rows_per_tile
        pltpu.sync_copy(z_hbm.at[pl.ds(r0, rows_per_tile)],
                        shared.at[pl.ds(r0, rows_per_tile)])
        plsc.subcore_barrier()

        base = c * per_core + s * per_tile

        def body(ci, carry):
            off = base + ci * blk
            pltpu.sync_copy(r_hbm.at[pl.ds(off, blk)], eidx)
            pltpu.sync_copy(ne_hbm.at[pl.ds(off, blk)], erow)
            pltpu.sync_copy(erow, shared.at[eidx], add=True)
            return carry

        lax.fori_loop(0, chunks, body, 0)
        plsc.subcore_barrier()

        @pl.when(c == 0)
        def _():
            pltpu.sync_copy(shared.at[pl.ds(r0, rows_per_tile)],
                            a0_hbm.at[pl.ds(r0, rows_per_tile)])

        @pl.when(c == 1)
        def _():
            pltpu.sync_copy(shared.at[pl.ds(r0, rows_per_tile)],
                            a1_hbm.at[pl.ds(r0, rows_per_tile)])

    return k(new_e, receivers, zeros_nd)


def _node_body(x_ref, a0_ref, a1_ref, w1x_ref, w1a_ref, b1_ref, w2_ref,
               b2_ref, out_ref):
    x = x_ref[...]
    agg = a0_ref[...] + a1_ref[...]
    h = b1_ref[...] + jnp.dot(x, w1x_ref[...],
                              preferred_element_type=jnp.float32)
    h = h + jnp.dot(agg, w1a_ref[...], preferred_element_type=jnp.float32)
    h = jnp.maximum(h, 0.0)
    out_ref[...] = x + b2_ref[...] + jnp.dot(
        h, w2_ref[...], preferred_element_type=jnp.float32)


def _node(x, a0, a1, w1x, w1a, b1, w2, b2):
    n, d = x.shape
    blk = 1000
    grid = n // blk
    assert grid * blk == n
    row = pl.BlockSpec((blk, d), lambda i: (i, 0))
    full = pl.BlockSpec((d, d), lambda i: (0, 0))
    vec = pl.BlockSpec((1, d), lambda i: (0, 0))
    return pl.pallas_call(
        _node_body,
        grid=(grid,),
        in_specs=[row, row, row, full, full, vec, full, vec],
        out_specs=row,
        out_shape=jax.ShapeDtypeStruct((n, d), jnp.float32),
    )(x, a0, a1, w1x, w1a, b1, w2, b2)


def kernel(x, edge_features, W1_e, b1_e, W2_e, b2_e,
           W1_n, b1_n, W2_n, b2_n, senders, receivers):
    n, d = x.shape
    senders = senders.astype(jnp.int32)
    receivers = receivers.astype(jnp.int32)
    w1_s, w1_r, w1_ef = W1_e[:d], W1_e[d:2 * d], W1_e[2 * d:]
    b1_e2 = b1_e.reshape(1, d)
    b2_e2 = b2_e.reshape(1, d)
    b1_n2 = b1_n.reshape(1, d)
    b2_n2 = b2_n.reshape(1, d)
    w1_nx, w1_na = W1_n[:d], W1_n[d:]
    n_pad = ((n + 8 * _NS - 1) // (8 * _NS)) * (8 * _NS)
    zeros_nd = jnp.zeros((n_pad, d), jnp.float32)

    xs, xr = _pre(x, w1_s, w1_r)
    gs, gr = _gather(xs, xr, senders, receivers)
    new_e = _edge(gs, gr, edge_features, w1_ef, b1_e2, W2_e, b2_e2)
    a0, a1 = _scatter(new_e, receivers, zeros_nd)
    new_x = _node(x, a0, a1, w1_nx, w1_na, b1_n2, W2_n, b2_n2)
    return (new_x, new_e)


# speedup vs baseline: 4.2562x; 1.1099x over previous
import functools

import jax
import jax.numpy as jnp
from jax import lax
from jax.experimental import pallas as pl
from jax.experimental.pallas import tpu as pltpu
from jax.experimental.pallas import tpu_sc as plsc

_NC = 2
_NS = 16
_NW = _NC * _NS


def _pre_body(x_ref, ws_ref, wr_ref, xs_ref, xr_ref):
    x = x_ref[...]
    xs_ref[...] = jnp.dot(x, ws_ref[...], preferred_element_type=jnp.float32)
    xr_ref[...] = jnp.dot(x, wr_ref[...], preferred_element_type=jnp.float32)


def _pre(x, w_s, w_r):
    n, d = x.shape
    out = jax.ShapeDtypeStruct((n, d), jnp.float32)
    return pl.pallas_call(
        _pre_body,
        out_shape=(out, out),
    )(x, w_s, w_r)


def _gather(xs, xr, senders, receivers):
    e = senders.shape[0]
    d = xs.shape[1]
    blk = 200
    per_w = e // _NW
    chunks = per_w // blk
    pairs = chunks // 2
    assert per_w * _NW == e and chunks * blk == per_w and pairs * 2 == chunks

    mesh = plsc.VectorSubcoreMesh(core_axis_name="c", subcore_axis_name="s")
    out = jax.ShapeDtypeStruct((e, d), jnp.float32)
    idx_t = pltpu.VMEM((blk,), jnp.int32)
    row_t = pltpu.VMEM((blk, d), jnp.float32)

    @functools.partial(
        pl.kernel,
        out_type=(out, out),
        mesh=mesh,
        scratch_types=[idx_t, idx_t, idx_t, idx_t, row_t, row_t, row_t, row_t,
                       pltpu.SemaphoreType.DMA, pltpu.SemaphoreType.DMA],
    )
    def k(xs_hbm, xr_hbm, s_hbm, r_hbm, gs_hbm, gr_hbm,
          sidx0, sidx1, ridx0, ridx1, srow0, srow1, rrow0, rrow1,
          gsem, wsem):
        wid = lax.axis_index("s") * _NC + lax.axis_index("c")
        base = wid * per_w
        sidx = (sidx0, sidx1)
        ridx = (ridx0, ridx1)
        srow = (srow0, srow1)
        rrow = (rrow0, rrow1)

        def issue(chunk, b):
            off = base + chunk * blk
            pltpu.sync_copy(s_hbm.at[pl.ds(off, blk)], sidx[b])
            pltpu.sync_copy(r_hbm.at[pl.ds(off, blk)], ridx[b])
            pltpu.async_copy(xs_hbm.at[sidx[b]], srow[b], gsem)
            pltpu.async_copy(xr_hbm.at[ridx[b]], rrow[b], gsem)

        def wait_gather(b):
            pltpu.make_async_copy(xs_hbm.at[sidx[b]], srow[b], gsem).wait()
            pltpu.make_async_copy(xr_hbm.at[ridx[b]], rrow[b], gsem).wait()

        def writeback(chunk, b):
            off = base + chunk * blk
            pltpu.async_copy(srow[b], gs_hbm.at[pl.ds(off, blk)], wsem)
            pltpu.async_copy(rrow[b], gr_hbm.at[pl.ds(off, blk)], wsem)

        def wait_writeback(chunk, b):
            off = base + chunk * blk
            pltpu.make_async_copy(
                srow[b], gs_hbm.at[pl.ds(off, blk)], wsem).wait()
            pltpu.make_async_copy(
                rrow[b], gr_hbm.at[pl.ds(off, blk)], wsem).wait()

        issue(0, 0)
        issue(1, 1)

        @pl.loop(0, pairs - 1)
        def _(p):
            c0 = 2 * p

            wait_gather(0)
            writeback(c0, 0)
            wait_gather(1)
            writeback(c0 + 1, 1)
            wait_writeback(c0, 0)
            issue(c0 + 2, 0)
            wait_writeback(c0 + 1, 1)
            issue(c0 + 3, 1)

        last = chunks - 2
        wait_gather(0)
        writeback(last, 0)
        wait_gather(1)
        writeback(last + 1, 1)
        wait_writeback(last, 0)
        wait_writeback(last + 1, 1)

    return k(xs, xr, senders, receivers)


def _edge_body(gs_ref, gr_ref, ef_ref, w1_ref, b1_ref, w2_ref, b2_ref, out_ref):
    ef = ef_ref[...]
    g = (gs_ref[...] + gr_ref[...]).astype(jnp.float32)
    h = g + b1_ref[...] + jnp.dot(
        ef.astype(jnp.bfloat16), w1_ref[...],
        preferred_element_type=jnp.float32)
    h = jnp.maximum(h, 0.0)
    out_ref[...] = ef + b2_ref[...] + jnp.dot(
        h.astype(jnp.bfloat16), w2_ref[...],
        preferred_element_type=jnp.float32)


def _edge(gs, gr, ef, w1, b1, w2, b2):
    e, d = ef.shape
    blk = 2000
    grid = e // blk
    assert grid * blk == e
    row = pl.BlockSpec((blk, d), lambda i: (i, 0))
    full = pl.BlockSpec((d, d), lambda i: (0, 0))
    vec = pl.BlockSpec((1, d), lambda i: (0, 0))
    return pl.pallas_call(
        _edge_body,
        grid=(grid,),
        in_specs=[row, row, row, full, vec, full, vec],
        out_specs=row,
        out_shape=jax.ShapeDtypeStruct((e, d), jnp.float32),
    )(gs, gr, ef, w1, b1, w2, b2)


def _scatter(new_e, receivers, zeros_nd):
    e, d = new_e.shape
    n = zeros_nd.shape[0]
    blk = 80
    per_core = e // _NC
    per_tile = per_core // _NS
    chunks = per_tile // blk
    rows_per_tile = n // _NS
    assert chunks * blk == per_tile and rows_per_tile * _NS == n
    assert chunks % 2 == 1
    assert rows_per_tile % 8 == 0

    mesh = plsc.VectorSubcoreMesh(core_axis_name="c", subcore_axis_name="s")
    out = jax.ShapeDtypeStruct((n, d), jnp.float32)
    idx_t = pltpu.VMEM((blk,), jnp.int32)
    row_t = pltpu.VMEM((blk, d), jnp.float32)

    @functools.partial(
        pl.kernel,
        out_type=(out, out),
        mesh=mesh,
        scratch_types=[
            pltpu.VMEM_SHARED((n, d), jnp.float32),
            idx_t, idx_t, row_t, row_t,
            pltpu.SemaphoreType.DMA, pltpu.SemaphoreType.DMA,
        ],
    )
    def k(ne_hbm, r_hbm, z_hbm, a0_hbm, a1_hbm, shared,
          eidx0, eidx1, erow0, erow1, lsem, asem):
        c = lax.axis_index("c")
        s = lax.axis_index("s")
        r0 = s * rows_per_tile
        pltpu.sync_copy(z_hbm.at[pl.ds(r0, rows_per_tile)],
                        shared.at[pl.ds(r0, rows_per_tile)])
        plsc.subcore_barrier()

        base = c * per_core + s * per_tile
        eidx = (eidx0, eidx1)
        erow = (erow0, erow1)

        def load(chunk, b):
            off = base + chunk * blk
            pltpu.async_copy(r_hbm.at[pl.ds(off, blk)], eidx[b], lsem)
            pltpu.async_copy(ne_hbm.at[pl.ds(off, blk)], erow[b], lsem)

        def wait_load(chunk, b):
            off = base + chunk * blk
            pltpu.make_async_copy(
                r_hbm.at[pl.ds(off, blk)], eidx[b], lsem).wait()
            pltpu.make_async_copy(
                ne_hbm.at[pl.ds(off, blk)], erow[b], lsem).wait()

        def add(b):
            pltpu.async_copy(erow[b], shared.at[eidx[b]], asem, add=True)

        def wait_add(b):
            pltpu.make_async_copy(
                erow[b], shared.at[eidx[b]], asem).wait()

        load(0, 0)
        load(1, 1)

        @pl.loop(0, chunks - 2, step=2)
        def _(ci):
            wait_load(ci, 0)
            add(0)
            wait_load(ci + 1, 1)
            add(1)
            wait_add(0)
            load(ci + 2, 0)
            wait_add(1)

            @pl.when(ci + 3 < chunks)
            def _():
                load(ci + 3, 1)

        wait_load(chunks - 1, 0)
        add(0)
        wait_add(0)

        plsc.subcore_barrier()

        @pl.when(c == 0)
        def _():
            pltpu.sync_copy(shared.at[pl.ds(r0, rows_per_tile)],
                            a0_hbm.at[pl.ds(r0, rows_per_tile)])

        @pl.when(c == 1)
        def _():
            pltpu.sync_copy(shared.at[pl.ds(r0, rows_per_tile)],
                            a1_hbm.at[pl.ds(r0, rows_per_tile)])

    return k(new_e, receivers, zeros_nd)


def _node_body(x_ref, a0_ref, a1_ref, w1x_ref, w1a_ref, b1_ref, w2_ref,
               b2_ref, out_ref):
    x = x_ref[...]
    agg = a0_ref[...] + a1_ref[...]
    h = b1_ref[...] + jnp.dot(x, w1x_ref[...],
                              preferred_element_type=jnp.float32)
    h = h + jnp.dot(agg, w1a_ref[...], preferred_element_type=jnp.float32)
    h = jnp.maximum(h, 0.0)
    out_ref[...] = x + b2_ref[...] + jnp.dot(
        h, w2_ref[...], preferred_element_type=jnp.float32)


def _node(x, a0, a1, w1x, w1a, b1, w2, b2):
    n, d = x.shape
    blk = 1000
    grid = n // blk
    assert grid * blk == n
    row = pl.BlockSpec((blk, d), lambda i: (i, 0))
    full = pl.BlockSpec((d, d), lambda i: (0, 0))
    vec = pl.BlockSpec((1, d), lambda i: (0, 0))
    return pl.pallas_call(
        _node_body,
        grid=(grid,),
        in_specs=[row, row, row, full, full, vec, full, vec],
        out_specs=row,
        out_shape=jax.ShapeDtypeStruct((n, d), jnp.float32),
    )(x, a0, a1, w1x, w1a, b1, w2, b2)


def kernel(x, edge_features, W1_e, b1_e, W2_e, b2_e,
           W1_n, b1_n, W2_n, b2_n, senders, receivers):
    n, d = x.shape
    senders = senders.astype(jnp.int32)
    receivers = receivers.astype(jnp.int32)
    w1_s, w1_r, w1_ef = W1_e[:d], W1_e[d:2 * d], W1_e[2 * d:]
    b1_e2 = b1_e.reshape(1, d)
    b2_e2 = b2_e.reshape(1, d)
    b1_n2 = b1_n.reshape(1, d)
    b2_n2 = b2_n.reshape(1, d)
    w1_nx, w1_na = W1_n[:d], W1_n[d:]
    n_pad = ((n + 8 * _NS - 1) // (8 * _NS)) * (8 * _NS)
    zeros_nd = jnp.zeros((n_pad, d), jnp.float32)

    xs, xr = _pre(x, w1_s, w1_r)
    gs, gr = _gather(xs, xr, senders, receivers)
    new_e = _edge(gs, gr, edge_features,
                  w1_ef.astype(jnp.bfloat16), b1_e2,
                  W2_e.astype(jnp.bfloat16), b2_e2)
    a0, a1 = _scatter(new_e, receivers, zeros_nd)
    new_x = _node(x, a0, a1, w1_nx, w1_na, b1_n2, W2_n, b2_n2)
    return (new_x, new_e)
